# baseline (speedup 1.0000x reference)
import functools

import jax
import jax.numpy as jnp
from jax import lax
from jax.experimental import pallas as pl
from jax.experimental.pallas import tpu as pltpu
from jax.experimental.pallas import tpu_sc as plsc

B = 16384
NUM_NUM = 13
F = 26
VOCAB = 100000
EMB = 32
HID = 128
OUT = 64
EPS = 1e-5

NC = 2
NS = 16
NW = NC * NS
TOT = B * F
PER_W = TOT // NW
IDX_ROWS = PER_W // 128
CHUNK_ROWS = 8
NCHUNK = IDX_ROWS // CHUNK_ROWS
CHUNK = CHUNK_ROWS * 128


def _sc_gather(tables_flat, idx2d):
    mesh = plsc.VectorSubcoreMesh(core_axis_name="c", subcore_axis_name="s")

    @functools.partial(
        pl.kernel,
        mesh=mesh,
        compiler_params=pltpu.CompilerParams(use_tc_tiling_on_sc=False),
        out_type=jax.ShapeDtypeStruct((TOT, EMB), jnp.float32),
        scratch_types=[
            pltpu.VMEM((CHUNK_ROWS, 128), jnp.int32),
            pltpu.VMEM((CHUNK, EMB), jnp.float32),
            pltpu.SemaphoreType.DMA,
        ],
    )
    def k(tab_hbm, idx_hbm, out_hbm, idx_v, rows_v, sem):
        wid = lax.axis_index("s") * NC + lax.axis_index("c")
        row0 = wid * IDX_ROWS

        def chunk_body(c, carry):
            base_row = row0 + c * CHUNK_ROWS
            pltpu.sync_copy(idx_hbm.at[pl.ds(base_row, CHUNK_ROWS)], idx_v)
            copies = []
            for j in range(CHUNK_ROWS):
                copies.append(
                    pltpu.async_copy(
                        tab_hbm.at[idx_v.at[j]],
                        rows_v.at[pl.ds(j * 128, 128)],
                        sem,
                    )
                )
            for cp in copies:
                cp.wait()
            out_base = wid * PER_W + c * CHUNK
            pltpu.sync_copy(rows_v, out_hbm.at[pl.ds(out_base, CHUNK)])
            return carry

        lax.fori_loop(0, NCHUNK, chunk_body, 0)

    return k(tables_flat, idx2d)


SB = 196
VROWS = SB * 128


def _relayout_body(in_ref, out_ref):
    a = in_ref[0]
    out_ref[...] = jnp.concatenate(
        [a[:, 0:128].T, a[:, 128:256].T, a[:, 256:384].T, a[:, 384:512].T],
        axis=1)


def _relayout(tables_t):
    return pl.pallas_call(
        _relayout_body,
        grid=(F, SB),
        in_specs=[
            pl.BlockSpec((1, EMB, 512), lambda f, s: (f, 0, s)),
        ],
        out_specs=pl.BlockSpec((128, 128), lambda f, s: (f * SB + s, 0)),
        out_shape=jax.ShapeDtypeStruct((F * VROWS, 128), jnp.float32),
    )(tables_t)


BT = 1024
T = B // BT


def _mlp1_body(xn_ref, xc_ref, w1n_ref, w1c_ref, b1_ref, h_ref, stats_ref):
    i = pl.program_id(0)
    h = jnp.dot(xc_ref[...], w1c_ref[...], preferred_element_type=jnp.float32)
    h = h + jnp.dot(xn_ref[...], w1n_ref[...], preferred_element_type=jnp.float32)
    h = jnp.maximum(h + b1_ref[...], 0.0)
    h_ref[...] = h

    @pl.when(i == 0)
    def _():
        stats_ref[...] = jnp.zeros_like(stats_ref)

    stats_ref[0:1, :] += jnp.sum(h, axis=0, keepdims=True)
    stats_ref[1:2, :] += jnp.sum(h * h, axis=0, keepdims=True)


def _mlp1(xn, xc, W1n, W1c, b1):
    return pl.pallas_call(
        _mlp1_body,
        grid=(T,),
        in_specs=[
            pl.BlockSpec((BT, NUM_NUM), lambda i: (i, 0)),
            pl.BlockSpec((BT, F * EMB), lambda i: (i, 0)),
            pl.BlockSpec((NUM_NUM, HID), lambda i: (0, 0)),
            pl.BlockSpec((F * EMB, HID), lambda i: (0, 0)),
            pl.BlockSpec((1, HID), lambda i: (0, 0)),
        ],
        out_specs=[
            pl.BlockSpec((BT, HID), lambda i: (i, 0)),
            pl.BlockSpec((2, HID), lambda i: (0, 0)),
        ],
        out_shape=[
            jax.ShapeDtypeStruct((B, HID), jnp.float32),
            jax.ShapeDtypeStruct((2, HID), jnp.float32),
        ],
    )(xn, xc, W1n, W1c, b1)


def _mlp2_body(h_ref, stats_ref, g_ref, be_ref, w2_ref, b2_ref, out_ref):
    stats = stats_ref[...]
    mean = stats[0:1, :] * (1.0 / B)
    var = stats[1:2, :] * (1.0 / B) - mean * mean
    inv = lax.rsqrt(var + EPS)
    scale = g_ref[...] * inv
    shift = be_ref[...] - mean * scale
    hn = h_ref[...] * scale + shift
    out_ref[...] = (
        jnp.dot(hn, w2_ref[...], preferred_element_type=jnp.float32) + b2_ref[...]
    )


def _mlp2(h, stats, gamma, beta, W2, b2):
    return pl.pallas_call(
        _mlp2_body,
        grid=(T,),
        in_specs=[
            pl.BlockSpec((BT, HID), lambda i: (i, 0)),
            pl.BlockSpec((2, HID), lambda i: (0, 0)),
            pl.BlockSpec((1, HID), lambda i: (0, 0)),
            pl.BlockSpec((1, HID), lambda i: (0, 0)),
            pl.BlockSpec((HID, OUT), lambda i: (0, 0)),
            pl.BlockSpec((1, OUT), lambda i: (0, 0)),
        ],
        out_specs=pl.BlockSpec((BT, OUT), lambda i: (i, 0)),
        out_shape=jax.ShapeDtypeStruct((B, OUT), jnp.float32),
    )(h, stats, gamma, beta, W2, b2)


def kernel(numerical_x, categorical_x, tables, W1, b1, gamma, beta, W2, b2):
    tables_flat = _relayout(tables.transpose(0, 2, 1)).reshape(F * VROWS * 4, EMB)
    cat = categorical_x
    idx = (4 * ((cat >> 9) * 128 + (cat & 127)) + ((cat >> 7) & 3)
           + (jnp.arange(F, dtype=jnp.int32) * (4 * VROWS))[None, :])
    idx2d = idx.reshape(TOT // 128, 128)
    xcat = _sc_gather(tables_flat, idx2d).reshape(B, F * EMB)

    W1n = W1[:NUM_NUM]
    W1c = W1[NUM_NUM:]
    h, stats = _mlp1(numerical_x, xcat, W1n, W1c, b1.reshape(1, HID))
    out = _mlp2(h, stats, gamma.reshape(1, HID), beta.reshape(1, HID), W2,
                b2.reshape(1, OUT))
    return out


# speedup vs baseline: 3.2111x; 3.2111x over previous
import functools

import jax
import jax.numpy as jnp
from jax import lax
from jax.experimental import pallas as pl
from jax.experimental.pallas import tpu as pltpu
from jax.experimental.pallas import tpu_sc as plsc

B = 16384
NUM_NUM = 13
F = 26
VOCAB = 100000
EMB = 32
HID = 128
OUT = 64
EPS = 1e-5

NC = 2
NS = 16
NW = NC * NS
TOT = B * F
PER_W = TOT // NW
IDX_ROWS = PER_W // 128
CHUNK_ROWS = 8
NCHUNK = IDX_ROWS // CHUNK_ROWS
CHUNK = CHUNK_ROWS * 128


def _sc_gather(tables_flat, idx2d):
    mesh = plsc.VectorSubcoreMesh(core_axis_name="c", subcore_axis_name="s")

    @functools.partial(
        pl.kernel,
        mesh=mesh,
        compiler_params=pltpu.CompilerParams(use_tc_tiling_on_sc=False),
        out_type=jax.ShapeDtypeStruct((TOT, EMB), jnp.float32),
        scratch_types=[
            pltpu.VMEM((CHUNK_ROWS, 128), jnp.int32),
            pltpu.VMEM((CHUNK, EMB), jnp.float32),
            pltpu.SemaphoreType.DMA,
        ],
    )
    def k(tab_hbm, idx_hbm, out_hbm, idx_v, rows_v, sem):
        wid = lax.axis_index("s") * NC + lax.axis_index("c")
        row0 = wid * IDX_ROWS

        def chunk_body(c, carry):
            base_row = row0 + c * CHUNK_ROWS
            pltpu.sync_copy(idx_hbm.at[pl.ds(base_row, CHUNK_ROWS)], idx_v)
            copies = []
            for j in range(CHUNK_ROWS):
                copies.append(
                    pltpu.async_copy(
                        tab_hbm.at[idx_v.at[j]],
                        rows_v.at[pl.ds(j * 128, 128)],
                        sem,
                    )
                )
            for cp in copies:
                cp.wait()
            out_base = wid * PER_W + c * CHUNK
            pltpu.sync_copy(rows_v, out_hbm.at[pl.ds(out_base, CHUNK)])
            return carry

        lax.fori_loop(0, NCHUNK, chunk_body, 0)

    return k(tables_flat, idx2d)


SB = 196
VROWS = SB * 128


RL_KS = 7
RL_STEPS = SB // RL_KS


def _relayout_body(in_ref, out_ref):
    a = in_ref[0]
    for s in range(RL_KS):
        base = 512 * s
        out_ref[pl.ds(128 * s, 128), :] = jnp.concatenate(
            [a[:, base:base + 128].T,
             a[:, base + 128:base + 256].T,
             a[:, base + 256:base + 384].T,
             a[:, base + 384:base + 512].T], axis=1)


def _relayout(tables_t):
    return pl.pallas_call(
        _relayout_body,
        grid=(F, RL_STEPS),
        in_specs=[
            pl.BlockSpec((1, EMB, 512 * RL_KS), lambda f, s: (f, 0, s)),
        ],
        out_specs=pl.BlockSpec((128 * RL_KS, 128),
                               lambda f, s: (f * RL_STEPS + s, 0)),
        out_shape=jax.ShapeDtypeStruct((F * VROWS, 128), jnp.float32),
    )(tables_t)


BT = 1024
T = B // BT


def _mlp1_body(xn_ref, xc_ref, w1n_ref, w1c_ref, b1_ref, h_ref, stats_ref):
    i = pl.program_id(0)
    h = jnp.dot(xc_ref[...], w1c_ref[...], preferred_element_type=jnp.float32)
    h = h + jnp.dot(xn_ref[...], w1n_ref[...], preferred_element_type=jnp.float32)
    h = jnp.maximum(h + b1_ref[...], 0.0)
    h_ref[...] = h

    @pl.when(i == 0)
    def _():
        stats_ref[...] = jnp.zeros_like(stats_ref)

    stats_ref[0:1, :] += jnp.sum(h, axis=0, keepdims=True)
    stats_ref[1:2, :] += jnp.sum(h * h, axis=0, keepdims=True)


def _mlp1(xn, xc, W1n, W1c, b1):
    return pl.pallas_call(
        _mlp1_body,
        grid=(T,),
        in_specs=[
            pl.BlockSpec((BT, NUM_NUM), lambda i: (i, 0)),
            pl.BlockSpec((BT, F * EMB), lambda i: (i, 0)),
            pl.BlockSpec((NUM_NUM, HID), lambda i: (0, 0)),
            pl.BlockSpec((F * EMB, HID), lambda i: (0, 0)),
            pl.BlockSpec((1, HID), lambda i: (0, 0)),
        ],
        out_specs=[
            pl.BlockSpec((BT, HID), lambda i: (i, 0)),
            pl.BlockSpec((2, HID), lambda i: (0, 0)),
        ],
        out_shape=[
            jax.ShapeDtypeStruct((B, HID), jnp.float32),
            jax.ShapeDtypeStruct((2, HID), jnp.float32),
        ],
    )(xn, xc, W1n, W1c, b1)


def _mlp2_body(h_ref, stats_ref, g_ref, be_ref, w2_ref, b2_ref, out_ref):
    stats = stats_ref[...]
    mean = stats[0:1, :] * (1.0 / B)
    var = stats[1:2, :] * (1.0 / B) - mean * mean
    inv = lax.rsqrt(var + EPS)
    scale = g_ref[...] * inv
    shift = be_ref[...] - mean * scale
    hn = h_ref[...] * scale + shift
    out_ref[...] = (
        jnp.dot(hn, w2_ref[...], preferred_element_type=jnp.float32) + b2_ref[...]
    )


def _mlp2(h, stats, gamma, beta, W2, b2):
    return pl.pallas_call(
        _mlp2_body,
        grid=(T,),
        in_specs=[
            pl.BlockSpec((BT, HID), lambda i: (i, 0)),
            pl.BlockSpec((2, HID), lambda i: (0, 0)),
            pl.BlockSpec((1, HID), lambda i: (0, 0)),
            pl.BlockSpec((1, HID), lambda i: (0, 0)),
            pl.BlockSpec((HID, OUT), lambda i: (0, 0)),
            pl.BlockSpec((1, OUT), lambda i: (0, 0)),
        ],
        out_specs=pl.BlockSpec((BT, OUT), lambda i: (i, 0)),
        out_shape=jax.ShapeDtypeStruct((B, OUT), jnp.float32),
    )(h, stats, gamma, beta, W2, b2)


def kernel(numerical_x, categorical_x, tables, W1, b1, gamma, beta, W2, b2):
    tables_flat = _relayout(tables.transpose(0, 2, 1)).reshape(F * VROWS * 4, EMB)
    cat = categorical_x
    idx = (4 * ((cat >> 9) * 128 + (cat & 127)) + ((cat >> 7) & 3)
           + (jnp.arange(F, dtype=jnp.int32) * (4 * VROWS))[None, :])
    idx2d = idx.reshape(TOT // 128, 128)
    xcat = _sc_gather(tables_flat, idx2d).reshape(B, F * EMB)

    W1n = W1[:NUM_NUM]
    W1c = W1[NUM_NUM:]
    h, stats = _mlp1(numerical_x, xcat, W1n, W1c, b1.reshape(1, HID))
    out = _mlp2(h, stats, gamma.reshape(1, HID), beta.reshape(1, HID), W2,
                b2.reshape(1, OUT))
    return out


# speedup vs baseline: 4.1505x; 1.2925x over previous
import functools

import jax
import jax.numpy as jnp
from jax import lax
from jax.experimental import pallas as pl
from jax.experimental.pallas import tpu as pltpu
from jax.experimental.pallas import tpu_sc as plsc

B = 16384
NUM_NUM = 13
F = 26
VOCAB = 100000
EMB = 32
HID = 128
OUT = 64
EPS = 1e-5

NC = 2
NS = 16
NW = NC * NS
TOT = B * F
PER_W = TOT // NW
IDX_ROWS = PER_W // 128
CHUNK_ROWS = 8
NCHUNK = IDX_ROWS // CHUNK_ROWS
CHUNK = CHUNK_ROWS * 128


def _sc_gather(tables_flat, idx2d):
    mesh = plsc.VectorSubcoreMesh(core_axis_name="c", subcore_axis_name="s")

    @functools.partial(
        pl.kernel,
        mesh=mesh,
        compiler_params=pltpu.CompilerParams(use_tc_tiling_on_sc=False),
        out_type=jax.ShapeDtypeStruct((TOT, EMB), jnp.float32),
        scratch_types=[
            pltpu.VMEM((CHUNK_ROWS, 128), jnp.int32),
            pltpu.VMEM((CHUNK, EMB), jnp.float32),
            pltpu.SemaphoreType.DMA,
        ],
    )
    def k(tab_hbm, idx_hbm, out_hbm, idx_v, rows_v, sem):
        wid = lax.axis_index("s") * NC + lax.axis_index("c")
        row0 = wid * IDX_ROWS

        def chunk_body(c, carry):
            base_row = row0 + c * CHUNK_ROWS
            pltpu.sync_copy(idx_hbm.at[pl.ds(base_row, CHUNK_ROWS)], idx_v)
            copies = []
            for j in range(CHUNK_ROWS):
                copies.append(
                    pltpu.async_copy(
                        tab_hbm.at[idx_v.at[j]],
                        rows_v.at[pl.ds(j * 128, 128)],
                        sem,
                    )
                )
            for cp in copies:
                cp.wait()
            out_base = wid * PER_W + c * CHUNK
            pltpu.sync_copy(rows_v, out_hbm.at[pl.ds(out_base, CHUNK)])
            return carry

        lax.fori_loop(0, NCHUNK, chunk_body, 0)

    return k(tables_flat, idx2d)


SB = 196
VROWS = SB * 128


RL_KS = 14
RL_STEPS = SB // RL_KS


def _relayout_body(in_ref, e_ref, out_ref):
    a = in_ref[0]
    e_sel = e_ref[...]
    for s in range(RL_KS):
        base = 512 * s
        if s % 2 == 0:
            out_ref[pl.ds(128 * s, 128), :] = jnp.concatenate(
                [a[:, base:base + 128].T,
                 a[:, base + 128:base + 256].T,
                 a[:, base + 256:base + 384].T,
                 a[:, base + 384:base + 512].T], axis=1)
        else:
            mx = jax.lax.dot_general(
                a[:, base:base + 128], e_sel[0],
                (((0,), (0,)), ((), ())), preferred_element_type=jnp.float32)
            for j in range(1, 4):
                mx = mx + jax.lax.dot_general(
                    a[:, base + 128 * j:base + 128 * (j + 1)], e_sel[j],
                    (((0,), (0,)), ((), ())),
                    preferred_element_type=jnp.float32)
            out_ref[pl.ds(128 * s, 128), :] = mx


def _relayout(tables_t):
    e_np = jnp.zeros((4, 32, 128), jnp.float32)
    for j in range(4):
        e_np = e_np.at[j, jnp.arange(32), 32 * j + jnp.arange(32)].set(1.0)
    return pl.pallas_call(
        _relayout_body,
        grid=(F, RL_STEPS),
        compiler_params=pltpu.CompilerParams(
            fuse_transposed_lhs_in_matmul=True),
        in_specs=[
            pl.BlockSpec((1, EMB, 512 * RL_KS), lambda f, s: (f, 0, s)),
            pl.BlockSpec((4, EMB, 128), lambda f, s: (0, 0, 0)),
        ],
        out_specs=pl.BlockSpec((128 * RL_KS, 128),
                               lambda f, s: (f * RL_STEPS + s, 0)),
        out_shape=jax.ShapeDtypeStruct((F * VROWS, 128), jnp.float32),
    )(tables_t, e_np)


BT = 1024
T = B // BT


def _mlp1_body(xn_ref, xc_ref, w1n_ref, w1c_ref, b1_ref, h_ref, stats_ref):
    i = pl.program_id(0)
    h = jnp.dot(xc_ref[...], w1c_ref[...], preferred_element_type=jnp.float32)
    h = h + jnp.dot(xn_ref[...], w1n_ref[...], preferred_element_type=jnp.float32)
    h = jnp.maximum(h + b1_ref[...], 0.0)
    h_ref[...] = h

    @pl.when(i == 0)
    def _():
        stats_ref[...] = jnp.zeros_like(stats_ref)

    stats_ref[0:1, :] += jnp.sum(h, axis=0, keepdims=True)
    stats_ref[1:2, :] += jnp.sum(h * h, axis=0, keepdims=True)


def _mlp1(xn, xc, W1n, W1c, b1):
    return pl.pallas_call(
        _mlp1_body,
        grid=(T,),
        in_specs=[
            pl.BlockSpec((BT, NUM_NUM), lambda i: (i, 0)),
            pl.BlockSpec((BT, F * EMB), lambda i: (i, 0)),
            pl.BlockSpec((NUM_NUM, HID), lambda i: (0, 0)),
            pl.BlockSpec((F * EMB, HID), lambda i: (0, 0)),
            pl.BlockSpec((1, HID), lambda i: (0, 0)),
        ],
        out_specs=[
            pl.BlockSpec((BT, HID), lambda i: (i, 0)),
            pl.BlockSpec((2, HID), lambda i: (0, 0)),
        ],
        out_shape=[
            jax.ShapeDtypeStruct((B, HID), jnp.float32),
            jax.ShapeDtypeStruct((2, HID), jnp.float32),
        ],
    )(xn, xc, W1n, W1c, b1)


def _mlp2_body(h_ref, stats_ref, g_ref, be_ref, w2_ref, b2_ref, out_ref):
    stats = stats_ref[...]
    mean = stats[0:1, :] * (1.0 / B)
    var = stats[1:2, :] * (1.0 / B) - mean * mean
    inv = lax.rsqrt(var + EPS)
    scale = g_ref[...] * inv
    shift = be_ref[...] - mean * scale
    hn = h_ref[...] * scale + shift
    out_ref[...] = (
        jnp.dot(hn, w2_ref[...], preferred_element_type=jnp.float32) + b2_ref[...]
    )


def _mlp2(h, stats, gamma, beta, W2, b2):
    return pl.pallas_call(
        _mlp2_body,
        grid=(T,),
        in_specs=[
            pl.BlockSpec((BT, HID), lambda i: (i, 0)),
            pl.BlockSpec((2, HID), lambda i: (0, 0)),
            pl.BlockSpec((1, HID), lambda i: (0, 0)),
            pl.BlockSpec((1, HID), lambda i: (0, 0)),
            pl.BlockSpec((HID, OUT), lambda i: (0, 0)),
            pl.BlockSpec((1, OUT), lambda i: (0, 0)),
        ],
        out_specs=pl.BlockSpec((BT, OUT), lambda i: (i, 0)),
        out_shape=jax.ShapeDtypeStruct((B, OUT), jnp.float32),
    )(h, stats, gamma, beta, W2, b2)


def kernel(numerical_x, categorical_x, tables, W1, b1, gamma, beta, W2, b2):
    tables_flat = _relayout(tables.transpose(0, 2, 1)).reshape(F * VROWS * 4, EMB)
    cat = categorical_x
    idx = (4 * ((cat >> 9) * 128 + (cat & 127)) + ((cat >> 7) & 3)
           + (jnp.arange(F, dtype=jnp.int32) * (4 * VROWS))[None, :])
    idx2d = idx.reshape(TOT // 128, 128)
    xcat = _sc_gather(tables_flat, idx2d).reshape(B, F * EMB)

    W1n = W1[:NUM_NUM]
    W1c = W1[NUM_NUM:]
    h, stats = _mlp1(numerical_x, xcat, W1n, W1c, b1.reshape(1, HID))
    out = _mlp2(h, stats, gamma.reshape(1, HID), beta.reshape(1, HID), W2,
                b2.reshape(1, OUT))
    return out
